# baseline (speedup 1.0000x reference)
import functools
import jax
import jax.numpy as jnp
from jax import lax
from jax.experimental import pallas as pl
from jax.experimental.pallas import tpu as pltpu
from jax.experimental.pallas import tpu_sc as plsc

B, D, H, W = 4, 96, 384, 384
DT = 80
DS = D - DT
HT = 32
NH = H // HT
NP = H * W
NR = NP // 128
NWK = 32
WPB = NWK // B
RW = NR // WPB
RC = 24
NCH = RW // RC
NL = 128 // 16

_mesh = plsc.VectorSubcoreMesh(core_axis_name="c", subcore_axis_name="s")


@functools.partial(
    pl.kernel,
    mesh=_mesh,
    out_type=jax.ShapeDtypeStruct((3, B * NR, 128), jnp.float32),
    scratch_types=[
        pltpu.VMEM((DS, RC, 128), jnp.float32),
        pltpu.VMEM((DS, RC, 128), jnp.float32),
        pltpu.VMEM((RC, 128), jnp.float32),
        pltpu.VMEM((RC, 128), jnp.float32),
        pltpu.VMEM((RC, 128), jnp.float32),
    ],
)
def _sc_partial(s_hbm, t_hbm, out_hbm, s_v, t_v, pd_v, pn_v, pt_v):
    wid = lax.axis_index("s") * 2 + lax.axis_index("c")
    b = wid // WPB
    lrow = (wid % WPB) * RW

    def _chunk(c, _):
        row0 = pl.multiple_of(lrow + c * RC, 8)
        grow0 = pl.multiple_of(wid * RW + c * RC, 8)
        for d in range(DS):
            pltpu.sync_copy(s_hbm.at[b * D + DT + d, pl.ds(row0, RC)],
                            s_v.at[d])
            pltpu.sync_copy(t_hbm.at[b * D + DT + d, pl.ds(row0, RC)],
                            t_v.at[d])

        def _row(r, _):
            for j in range(NL):
                sl = pl.ds(j * 16, 16)
                accd = accn = acct = None
                for d in range(DS):
                    sv = s_v[d, r, sl]
                    tv = t_v[d, r, sl]
                    if d == 0:
                        accd, accn, acct = sv * tv, sv * sv, tv * tv
                    else:
                        accd = accd + sv * tv
                        accn = accn + sv * sv
                        acct = acct + tv * tv
                pd_v[r, sl] = accd
                pn_v[r, sl] = accn
                pt_v[r, sl] = acct
            return 0

        lax.fori_loop(0, RC, _row, 0)
        pltpu.sync_copy(pd_v, out_hbm.at[0, pl.ds(grow0, RC)])
        pltpu.sync_copy(pn_v, out_hbm.at[1, pl.ds(grow0, RC)])
        pltpu.sync_copy(pt_v, out_hbm.at[2, pl.ds(grow0, RC)])
        return 0

    lax.fori_loop(0, NCH, _chunk, 0)


def _tc_body(s_ref, t_ref, d_ref, n_ref, t2_ref):
    dot = ns = nt = None
    for d in range(DT):
        sd = s_ref[0, d]
        td = t_ref[0, d]
        if d == 0:
            dot, ns, nt = sd * td, sd * sd, td * td
        else:
            dot = dot + sd * td
            ns = ns + sd * sd
            nt = nt + td * td
    d_ref[0] = dot
    n_ref[0] = ns
    t2_ref[0] = nt


def _combine_body(td_ref, tn_ref, tt_ref, sc_ref, m_ref, x_ref,
                  sum_ref, cnt_ref):
    b = pl.program_id(0)

    @pl.when(b == 0)
    def _init():
        sum_ref[...] = jnp.zeros_like(sum_ref)
        cnt_ref[...] = jnp.zeros_like(cnt_ref)

    dot = td_ref[0] + sc_ref[0, 0]
    ns = tn_ref[0] + sc_ref[1, 0]
    nt = tt_ref[0] + sc_ref[2, 0]
    denom = jnp.maximum(jnp.sqrt(ns) * jnp.sqrt(nt), 1e-8)
    loss_map = 1.0 - dot / denom
    valid = (x_ref[0] != 0.0) & (m_ref[0] == 0)
    vf = valid.astype(jnp.float32)
    sum_ref[...] += jnp.sum(loss_map * vf, keepdims=True).reshape(1, 1)
    cnt_ref[...] += jnp.sum(vf, keepdims=True).reshape(1, 1)


def kernel(student_feats, teacher_feats, mask, original_x):
    m = mask.astype(jnp.int8)
    x = original_x.reshape(B, H, W)
    s_flat = student_feats.reshape(B * D, NR, 128)
    t_flat = teacher_feats.reshape(B * D, NR, 128)

    sc_pm = _sc_partial(s_flat, t_flat)
    sc_pm = sc_pm.reshape(3, B, H, W)

    feat_spec = pl.BlockSpec((1, DT, HT, W), lambda b, h: (b, 0, h, 0))
    map_spec = pl.BlockSpec((1, HT, W), lambda b, h: (b, h, 0))
    tcd, tcn, tct = pl.pallas_call(
        _tc_body,
        grid=(B, NH),
        in_specs=[feat_spec, feat_spec],
        out_specs=[map_spec, map_spec, map_spec],
        out_shape=[jax.ShapeDtypeStruct((B, H, W), jnp.float32)] * 3,
        compiler_params=pltpu.CompilerParams(
            dimension_semantics=("arbitrary", "arbitrary"),
        ),
    )(student_feats, teacher_feats)

    full_spec = pl.BlockSpec((1, H, W), lambda b: (b, 0, 0))
    sums, cnts = pl.pallas_call(
        _combine_body,
        grid=(B,),
        in_specs=[
            full_spec, full_spec, full_spec,
            pl.BlockSpec((3, 1, H, W), lambda b: (0, b, 0, 0)),
            full_spec, full_spec,
        ],
        out_specs=[
            pl.BlockSpec((1, 1), lambda b: (0, 0)),
            pl.BlockSpec((1, 1), lambda b: (0, 0)),
        ],
        out_shape=[
            jax.ShapeDtypeStruct((1, 1), jnp.float32),
            jax.ShapeDtypeStruct((1, 1), jnp.float32),
        ],
        compiler_params=pltpu.CompilerParams(
            dimension_semantics=("arbitrary",),
        ),
    )(tcd, tcn, tct, sc_pm, m, x)

    return sums[0, 0] / cnts[0, 0]


# speedup vs baseline: 4.9281x; 4.9281x over previous
import jax
import jax.numpy as jnp
from jax.experimental import pallas as pl
from jax.experimental.pallas import tpu as pltpu

B, D, H, W = 4, 96, 384, 384
HT = 32
NH = H // HT


def _body(s_ref, t_ref, m_ref, x_ref, sum_ref, cnt_ref):
    b = pl.program_id(0)
    h = pl.program_id(1)

    @pl.when((b == 0) & (h == 0))
    def _init():
        sum_ref[...] = jnp.zeros_like(sum_ref)
        cnt_ref[...] = jnp.zeros_like(cnt_ref)

    dot = ns = nt = None
    for d in range(D):
        sd = s_ref[0, d]
        td = t_ref[0, d]
        if d == 0:
            dot, ns, nt = sd * td, sd * sd, td * td
        else:
            dot = dot + sd * td
            ns = ns + sd * sd
            nt = nt + td * td

    denom = jnp.maximum(jnp.sqrt(ns) * jnp.sqrt(nt), 1e-8)
    loss_map = 1.0 - dot / denom
    valid = (x_ref[0] != 0.0) & (m_ref[0] == 0)
    vf = valid.astype(jnp.float32)
    sum_ref[...] += jnp.sum(loss_map * vf, keepdims=True).reshape(1, 1)
    cnt_ref[...] += jnp.sum(vf, keepdims=True).reshape(1, 1)


def kernel(student_feats, teacher_feats, mask, original_x):
    m = mask.astype(jnp.int8)
    x = original_x.reshape(B, H, W)

    feat_spec = pl.BlockSpec((1, D, HT, W), lambda b, h: (b, 0, h, 0))
    pix_spec = pl.BlockSpec((1, HT, W), lambda b, h: (b, h, 0))

    sums, cnts = pl.pallas_call(
        _body,
        grid=(B, NH),
        in_specs=[feat_spec, feat_spec, pix_spec, pix_spec],
        out_specs=[
            pl.BlockSpec((1, 1), lambda b, h: (0, 0)),
            pl.BlockSpec((1, 1), lambda b, h: (0, 0)),
        ],
        out_shape=[
            jax.ShapeDtypeStruct((1, 1), jnp.float32),
            jax.ShapeDtypeStruct((1, 1), jnp.float32),
        ],
        compiler_params=pltpu.CompilerParams(
            dimension_semantics=("arbitrary", "arbitrary"),
        ),
    )(student_feats, teacher_feats, m, x)

    return sums[0, 0] / cnts[0, 0]
